# Initial kernel scaffold; baseline (speedup 1.0000x reference)
import jax
import jax.numpy as jnp
from jax.experimental import pallas as pl


def kernel(probs, mask, prev_log_beam_prob):
    raise NotImplementedError("write your pallas kernel here")


# speedup vs baseline: 1.4771x; 1.4771x over previous
import jax
import jax.numpy as jnp
from jax.experimental import pallas as pl

BW = 8
BS = 128
NN = 8192
RPB = 8
NB = BS // RPB
NC = NN // 128
NG = BW * NC


def _topk_body(p_ref, pv_ref, sel_ref, bbi_ref, logp_ref):
    o = pl.program_id(0)
    x = p_ref[...].reshape(BW, RPB, NC, 128)
    xt = jnp.transpose(x, (1, 0, 2, 3))
    pv = pv_ref[...].reshape(RPB, BW)
    v = jnp.log(xt) + pv[:, :, None, None]
    varr = v.reshape(RPB, NG, 128)
    gmax = jnp.max(varr, axis=2)

    gidx = jax.lax.broadcasted_iota(jnp.int32, (RPB, NG), 1)
    BIG = jnp.int32(1 << 30)
    NEG = jnp.float32(-jnp.inf)

    oh = jnp.zeros((RPB, BW, NG), jnp.float32)
    kio3 = jax.lax.broadcasted_iota(jnp.int32, (RPB, BW, NG), 1)
    g3 = jax.lax.broadcasted_iota(jnp.int32, (RPB, BW, NG), 2)
    jgmat = jnp.zeros((RPB, BW), jnp.int32)
    kio2 = jax.lax.broadcasted_iota(jnp.int32, (RPB, BW), 1)
    for k in range(BW):
        m = jnp.max(gmax, axis=1, keepdims=True)
        jg = jnp.min(jnp.where(gmax == m, gidx, BIG), axis=1, keepdims=True)
        gmax = jnp.where(gidx == jg, NEG, gmax)
        oh = jnp.where((kio3 == k) & (g3 == jg[:, :, None]), 1.0, oh)
        jgmat = jnp.where(kio2 == k, jg, jgmat)

    cand = jax.lax.dot_general(oh, varr, (((2,), (1,)), ((0,), (0,))),
                               precision=jax.lax.Precision.HIGHEST,
                               preferred_element_type=jnp.float32)
    lio = jax.lax.broadcasted_iota(jnp.int32, (RPB, BW, 128), 2)
    cidx = jgmat[:, :, None] * 128 + lio

    vals = jnp.zeros((RPB, BW), jnp.float32)
    inds = jnp.zeros((RPB, BW), jnp.int32)
    for k in range(BW):
        m = jnp.max(jnp.max(cand, axis=2), axis=1, keepdims=True)
        gi = jnp.min(jnp.min(jnp.where(cand == m[:, :, None], cidx, BIG),
                             axis=2), axis=1, keepdims=True)
        cand = jnp.where(cidx == gi[:, :, None], NEG, cand)
        vals = jnp.where(kio2 == k, m, vals)
        inds = jnp.where(kio2 == k, gi, inds)

    valsT = vals.T
    indsT = inds.T
    sel = indsT % NN
    par = indsT // NN
    rio = jax.lax.broadcasted_iota(jnp.int32, (BW, RPB), 1)
    sel_ref[...] = sel.reshape(1, BW, RPB)
    bbi_ref[...] = (par * BS + (o * RPB + rio)).reshape(1, BW, RPB)
    logp_ref[...] = valsT.reshape(1, BW, RPB)


@jax.jit
def kernel(probs, mask, prev_log_beam_prob):
    del mask
    p5 = probs.reshape(BW, NB, RPB, NC, 128)
    prevT = prev_log_beam_prob.reshape(BW, NB, RPB).transpose(1, 2, 0)
    out_sd = jax.ShapeDtypeStruct((NB, BW, RPB), jnp.int32)
    out_sf = jax.ShapeDtypeStruct((NB, BW, RPB), jnp.float32)
    sel, bbi, logp = pl.pallas_call(
        _topk_body,
        grid=(NB,),
        in_specs=[
            pl.BlockSpec((BW, 1, RPB, NC, 128), lambda o: (0, o, 0, 0, 0)),
            pl.BlockSpec((1, RPB, BW), lambda o: (o, 0, 0)),
        ],
        out_specs=[
            pl.BlockSpec((1, BW, RPB), lambda o: (o, 0, 0)),
            pl.BlockSpec((1, BW, RPB), lambda o: (o, 0, 0)),
            pl.BlockSpec((1, BW, RPB), lambda o: (o, 0, 0)),
        ],
        out_shape=[out_sd, out_sd, out_sf],
    )(p5, prevT)
    return (sel.transpose(1, 0, 2).reshape(-1),
            bbi.transpose(1, 0, 2).reshape(-1),
            logp.transpose(1, 0, 2).reshape(-1))


# speedup vs baseline: 1.9597x; 1.3268x over previous
import jax
import jax.numpy as jnp
from jax.experimental import pallas as pl

BW = 8
BS = 128
NN = 8192
RPB = 16
NB = BS // RPB
NC = NN // 128
NG = BW * NC


def _topk_body(p_ref, pv_ref, sel_ref, bbi_ref, logp_ref):
    o = pl.program_id(0)
    x = p_ref[...].reshape(BW, RPB, NC, 128)
    xt = jnp.transpose(x, (1, 0, 2, 3))
    pv = pv_ref[...].reshape(RPB, BW)
    v = jnp.log(xt) + pv[:, :, None, None]
    varr = v.reshape(RPB, NG, 128)
    gmax = jnp.max(varr, axis=2)

    gidx = jax.lax.broadcasted_iota(jnp.int32, (RPB, NG), 1)
    BIG = jnp.int32(1 << 30)
    NEG = jnp.float32(-jnp.inf)

    oh = jnp.zeros((RPB, BW, NG), jnp.float32)
    kio3 = jax.lax.broadcasted_iota(jnp.int32, (RPB, BW, NG), 1)
    g3 = jax.lax.broadcasted_iota(jnp.int32, (RPB, BW, NG), 2)
    jgmat = jnp.zeros((RPB, BW), jnp.int32)
    kio2 = jax.lax.broadcasted_iota(jnp.int32, (RPB, BW), 1)
    for k in range(BW):
        m = jnp.max(gmax, axis=1, keepdims=True)
        jg = jnp.min(jnp.where(gmax == m, gidx, BIG), axis=1, keepdims=True)
        gmax = jnp.where(gidx == jg, NEG, gmax)
        oh = jnp.where((kio3 == k) & (g3 == jg[:, :, None]), 1.0, oh)
        jgmat = jnp.where(kio2 == k, jg, jgmat)

    cand = jax.lax.dot_general(oh, varr, (((2,), (1,)), ((0,), (0,))),
                               precision=jax.lax.Precision.HIGHEST,
                               preferred_element_type=jnp.float32)
    lio = jax.lax.broadcasted_iota(jnp.int32, (RPB, BW, 128), 2)
    cidx = jgmat[:, :, None] * 128 + lio

    vals = jnp.zeros((RPB, BW), jnp.float32)
    inds = jnp.zeros((RPB, BW), jnp.int32)
    for k in range(BW):
        m = jnp.max(jnp.max(cand, axis=2), axis=1, keepdims=True)
        gi = jnp.min(jnp.min(jnp.where(cand == m[:, :, None], cidx, BIG),
                             axis=2), axis=1, keepdims=True)
        cand = jnp.where(cidx == gi[:, :, None], NEG, cand)
        vals = jnp.where(kio2 == k, m, vals)
        inds = jnp.where(kio2 == k, gi, inds)

    valsT = vals.T
    indsT = inds.T
    sel = indsT % NN
    par = indsT // NN
    rio = jax.lax.broadcasted_iota(jnp.int32, (BW, RPB), 1)
    sel_ref[...] = sel.reshape(1, BW, RPB)
    bbi_ref[...] = (par * BS + (o * RPB + rio)).reshape(1, BW, RPB)
    logp_ref[...] = valsT.reshape(1, BW, RPB)


@jax.jit
def kernel(probs, mask, prev_log_beam_prob):
    del mask
    p5 = probs.reshape(BW, NB, RPB, NC, 128)
    prevT = prev_log_beam_prob.reshape(BW, NB, RPB).transpose(1, 2, 0)
    out_sd = jax.ShapeDtypeStruct((NB, BW, RPB), jnp.int32)
    out_sf = jax.ShapeDtypeStruct((NB, BW, RPB), jnp.float32)
    sel, bbi, logp = pl.pallas_call(
        _topk_body,
        grid=(NB,),
        in_specs=[
            pl.BlockSpec((BW, 1, RPB, NC, 128), lambda o: (0, o, 0, 0, 0)),
            pl.BlockSpec((1, RPB, BW), lambda o: (o, 0, 0)),
        ],
        out_specs=[
            pl.BlockSpec((1, BW, RPB), lambda o: (o, 0, 0)),
            pl.BlockSpec((1, BW, RPB), lambda o: (o, 0, 0)),
            pl.BlockSpec((1, BW, RPB), lambda o: (o, 0, 0)),
        ],
        out_shape=[out_sd, out_sd, out_sf],
    )(p5, prevT)
    return (sel.transpose(1, 0, 2).reshape(-1),
            bbi.transpose(1, 0, 2).reshape(-1),
            logp.transpose(1, 0, 2).reshape(-1))


# speedup vs baseline: 2.2652x; 1.1559x over previous
import jax
import jax.numpy as jnp
from jax.experimental import pallas as pl

BW = 8
BS = 128
NN = 8192
RPB = 32
NB = BS // RPB
NC = NN // 128
NG = BW * NC


def _topk_body(p_ref, pv_ref, sel_ref, bbi_ref, logp_ref):
    o = pl.program_id(0)
    x = p_ref[...].reshape(BW, RPB, NC, 128)
    xt = jnp.transpose(x, (1, 0, 2, 3))
    pv = pv_ref[...].reshape(RPB, BW)
    v = jnp.log(xt) + pv[:, :, None, None]
    varr = v.reshape(RPB, NG, 128)
    gmax = jnp.max(varr, axis=2)

    gidx = jax.lax.broadcasted_iota(jnp.int32, (RPB, NG), 1)
    BIG = jnp.int32(1 << 30)
    NEG = jnp.float32(-jnp.inf)

    oh = jnp.zeros((RPB, BW, NG), jnp.float32)
    kio3 = jax.lax.broadcasted_iota(jnp.int32, (RPB, BW, NG), 1)
    g3 = jax.lax.broadcasted_iota(jnp.int32, (RPB, BW, NG), 2)
    jgmat = jnp.zeros((RPB, BW), jnp.int32)
    kio2 = jax.lax.broadcasted_iota(jnp.int32, (RPB, BW), 1)
    for k in range(BW):
        m = jnp.max(gmax, axis=1, keepdims=True)
        jg = jnp.min(jnp.where(gmax == m, gidx, BIG), axis=1, keepdims=True)
        gmax = jnp.where(gidx == jg, NEG, gmax)
        oh = jnp.where((kio3 == k) & (g3 == jg[:, :, None]), 1.0, oh)
        jgmat = jnp.where(kio2 == k, jg, jgmat)

    cand = jax.lax.dot_general(oh, varr, (((2,), (1,)), ((0,), (0,))),
                               precision=jax.lax.Precision.HIGHEST,
                               preferred_element_type=jnp.float32)
    lio = jax.lax.broadcasted_iota(jnp.int32, (RPB, BW, 128), 2)
    cidx = jgmat[:, :, None] * 128 + lio

    vals = jnp.zeros((RPB, BW), jnp.float32)
    inds = jnp.zeros((RPB, BW), jnp.int32)
    for k in range(BW):
        m = jnp.max(jnp.max(cand, axis=2), axis=1, keepdims=True)
        gi = jnp.min(jnp.min(jnp.where(cand == m[:, :, None], cidx, BIG),
                             axis=2), axis=1, keepdims=True)
        cand = jnp.where(cidx == gi[:, :, None], NEG, cand)
        vals = jnp.where(kio2 == k, m, vals)
        inds = jnp.where(kio2 == k, gi, inds)

    valsT = vals.T
    indsT = inds.T
    sel = indsT % NN
    par = indsT // NN
    rio = jax.lax.broadcasted_iota(jnp.int32, (BW, RPB), 1)
    sel_ref[...] = sel.reshape(1, BW, RPB)
    bbi_ref[...] = (par * BS + (o * RPB + rio)).reshape(1, BW, RPB)
    logp_ref[...] = valsT.reshape(1, BW, RPB)


@jax.jit
def kernel(probs, mask, prev_log_beam_prob):
    del mask
    p5 = probs.reshape(BW, NB, RPB, NC, 128)
    prevT = prev_log_beam_prob.reshape(BW, NB, RPB).transpose(1, 2, 0)
    out_sd = jax.ShapeDtypeStruct((NB, BW, RPB), jnp.int32)
    out_sf = jax.ShapeDtypeStruct((NB, BW, RPB), jnp.float32)
    sel, bbi, logp = pl.pallas_call(
        _topk_body,
        grid=(NB,),
        in_specs=[
            pl.BlockSpec((BW, 1, RPB, NC, 128), lambda o: (0, o, 0, 0, 0)),
            pl.BlockSpec((1, RPB, BW), lambda o: (o, 0, 0)),
        ],
        out_specs=[
            pl.BlockSpec((1, BW, RPB), lambda o: (o, 0, 0)),
            pl.BlockSpec((1, BW, RPB), lambda o: (o, 0, 0)),
            pl.BlockSpec((1, BW, RPB), lambda o: (o, 0, 0)),
        ],
        out_shape=[out_sd, out_sd, out_sf],
    )(p5, prevT)
    return (sel.transpose(1, 0, 2).reshape(-1),
            bbi.transpose(1, 0, 2).reshape(-1),
            logp.transpose(1, 0, 2).reshape(-1))


# speedup vs baseline: 2.2689x; 1.0016x over previous
import jax
import jax.numpy as jnp
from jax.experimental import pallas as pl

BW = 8
BS = 128
NN = 8192
RPB = 32
NB = BS // RPB
NC = NN // 128
NG = BW * NC


def _topk_body(p_ref, pv_ref, sel_ref, bbi_ref, logp_ref):
    o = pl.program_id(0)
    x = p_ref[...].reshape(BW, RPB, NC, 128)
    xt = jnp.transpose(x, (1, 0, 2, 3))
    pv = pv_ref[...].reshape(RPB, BW)
    v = jnp.log(xt) + pv[:, :, None, None]
    varr = v.reshape(RPB, NG, 128)
    gmax = jnp.max(varr, axis=2)

    gidx = jax.lax.broadcasted_iota(jnp.int32, (RPB, NG), 1)
    BIG = jnp.int32(1 << 30)
    NEG = jnp.float32(-jnp.inf)

    g3 = jax.lax.broadcasted_iota(jnp.int32, (RPB, BW, NG), 2)
    jgmat = jnp.zeros((RPB, BW), jnp.int32)
    kio2 = jax.lax.broadcasted_iota(jnp.int32, (RPB, BW), 1)
    for k in range(BW):
        m = jnp.max(gmax, axis=1, keepdims=True)
        jg = jnp.min(jnp.where(gmax == m, gidx, BIG), axis=1, keepdims=True)
        gmax = jnp.where(gidx == jg, NEG, gmax)
        jgmat = jnp.where(kio2 == k, jg, jgmat)
    oh = (g3 == jgmat[:, :, None]).astype(jnp.float32)

    cand = jax.lax.dot_general(oh, varr, (((2,), (1,)), ((0,), (0,))),
                               precision=jax.lax.Precision.HIGHEST,
                               preferred_element_type=jnp.float32)
    lio = jax.lax.broadcasted_iota(jnp.int32, (RPB, BW, 128), 2)
    cidx = jgmat[:, :, None] * 128 + lio

    vals = jnp.zeros((RPB, BW), jnp.float32)
    inds = jnp.zeros((RPB, BW), jnp.int32)
    for k in range(BW):
        m = jnp.max(jnp.max(cand, axis=2), axis=1, keepdims=True)
        gi = jnp.min(jnp.min(jnp.where(cand == m[:, :, None], cidx, BIG),
                             axis=2), axis=1, keepdims=True)
        cand = jnp.where(cidx == gi[:, :, None], NEG, cand)
        vals = jnp.where(kio2 == k, m, vals)
        inds = jnp.where(kio2 == k, gi, inds)

    valsT = vals.T
    indsT = inds.T
    sel = indsT % NN
    par = indsT // NN
    rio = jax.lax.broadcasted_iota(jnp.int32, (BW, RPB), 1)
    sel_ref[...] = sel.reshape(1, BW, RPB)
    bbi_ref[...] = (par * BS + (o * RPB + rio)).reshape(1, BW, RPB)
    logp_ref[...] = valsT.reshape(1, BW, RPB)


@jax.jit
def kernel(probs, mask, prev_log_beam_prob):
    del mask
    p5 = probs.reshape(BW, NB, RPB, NC, 128)
    prevT = prev_log_beam_prob.reshape(BW, NB, RPB).transpose(1, 2, 0)
    out_sd = jax.ShapeDtypeStruct((NB, BW, RPB), jnp.int32)
    out_sf = jax.ShapeDtypeStruct((NB, BW, RPB), jnp.float32)
    sel, bbi, logp = pl.pallas_call(
        _topk_body,
        grid=(NB,),
        in_specs=[
            pl.BlockSpec((BW, 1, RPB, NC, 128), lambda o: (0, o, 0, 0, 0)),
            pl.BlockSpec((1, RPB, BW), lambda o: (o, 0, 0)),
        ],
        out_specs=[
            pl.BlockSpec((1, BW, RPB), lambda o: (o, 0, 0)),
            pl.BlockSpec((1, BW, RPB), lambda o: (o, 0, 0)),
            pl.BlockSpec((1, BW, RPB), lambda o: (o, 0, 0)),
        ],
        out_shape=[out_sd, out_sd, out_sf],
    )(p5, prevT)
    return (sel.transpose(1, 0, 2).reshape(-1),
            bbi.transpose(1, 0, 2).reshape(-1),
            logp.transpose(1, 0, 2).reshape(-1))
